# baseline (speedup 1.0000x reference)
import functools
import math

import jax
import jax.numpy as jnp
from jax import lax
from jax.experimental import pallas as pl
from jax.experimental.pallas import tpu as pltpu
from jax.experimental.pallas import tpu_sc as plsc

_VOCAB = 1000000
_EMB = 64
_B = 4096
_L = 200

_info = plsc.get_sparse_core_info()
_NC = _info.num_cores
_NS = _info.num_subcores
_NW = _NC * _NS
_CHUNK = 128
_CPW = _L
_NBUF = 4

_SCALE = math.sqrt(_EMB)

_LT = _L // 8
_ET = _EMB // 8


def _body(tok_hbm, table_hbm, out_hbm, idx_v, rows, buf, gsem, osem):
    wid = lax.axis_index("s") * _NC + lax.axis_index("c")
    pltpu.sync_copy(tok_hbm.at[:, wid], idx_v)
    iota = lax.iota(jnp.int32, 16)

    def gstart(j, b):
        pltpu.async_copy(table_hbm.at[idx_v.at[j // 8, j % 8]], rows[b], gsem[b])

    def gwait(j, b):
        pltpu.make_async_copy(
            table_hbm.at[idx_v.at[j // 8, j % 8]], rows[b], gsem[b]
        ).wait()

    def ostart(j, b):
        pltpu.async_copy(buf[b], out_hbm.at[pl.ds(j * _ET, _ET), wid], osem[b])

    def owait(j, b):
        pltpu.make_async_copy(
            buf[b], out_hbm.at[pl.ds(j * _ET, _ET), wid], osem[b]
        ).wait()

    def transpose_scale(b):
        @pl.loop(0, _EMB)
        def _(e):
            col_idx = jnp.full((16,), 0, jnp.int32) + e
            for kg in range(_CHUNK // 16):
                row_idx = kg * 16 + iota
                v = plsc.load_gather(rows[b], [row_idx, col_idx])
                buf[b][e // 8, e % 8, pl.ds(kg * 16, 16)] = v * _SCALE

    gstart(0, 0)
    gstart(1, 1)

    for b in range(_NBUF):
        j = b
        if j >= 2:
            owait(j - 2, (b - 2) % _NBUF)
        gstart(j + 2, (b + 2) % _NBUF)
        gwait(j, b)
        transpose_scale(b)
        ostart(j, b)

    @pl.loop(0, (_CPW - 2 * _NBUF) // _NBUF)
    def _(r):
        j0 = _NBUF + r * _NBUF
        for b in range(_NBUF):
            j = j0 + b
            owait(j - 2, (b - 2) % _NBUF)
            gstart(j + 2, (b + 2) % _NBUF)
            gwait(j, b)
            transpose_scale(b)
            ostart(j, b)

    for b in range(_NBUF):
        j = _CPW - _NBUF + b
        owait(j - 2, (b - 2) % _NBUF)
        if j + 2 < _CPW:
            gstart(j + 2, (b + 2) % _NBUF)
        gwait(j, b)
        transpose_scale(b)
        ostart(j, b)

    owait(_CPW - 2, (_NBUF - 2) % _NBUF)
    owait(_CPW - 1, _NBUF - 1)


_mesh = plsc.VectorSubcoreMesh(core_axis_name="c", subcore_axis_name="s")

_gather = functools.partial(
    pl.kernel,
    mesh=_mesh,
    out_type=jax.ShapeDtypeStruct((_L * _ET, _NW, 8, _CHUNK), jnp.float32),
    scratch_types=[
        pltpu.VMEM((_LT, 8, _CHUNK), jnp.int32),
        [pltpu.VMEM((_CHUNK, _EMB), jnp.float32) for _ in range(_NBUF)],
        [pltpu.VMEM((_ET, 8, _CHUNK), jnp.float32) for _ in range(_NBUF)],
        [pltpu.SemaphoreType.DMA for _ in range(_NBUF)],
        [pltpu.SemaphoreType.DMA for _ in range(_NBUF)],
    ],
    compiler_params=pltpu.CompilerParams(
        use_tc_tiling_on_sc=False, needs_layout_passes=False
    ),
)(_body)


def kernel(tokens, table):
    tok = (
        tokens.astype(jnp.int32)
        .T.reshape(_LT, 8, _NW, _CHUNK)
        .transpose(0, 2, 1, 3)
    )
    out = _gather(tok, table)
    return (
        out.reshape(_L, _ET, _NW, 8, _CHUNK)
        .transpose(2, 4, 0, 1, 3)
        .reshape(_B, _L, _EMB)
    )


# speedup vs baseline: 1.5170x; 1.5170x over previous
import functools
import math

import jax
import jax.numpy as jnp
from jax import lax
from jax.experimental import pallas as pl
from jax.experimental.pallas import tpu as pltpu
from jax.experimental.pallas import tpu_sc as plsc

_VOCAB = 1000000
_EMB = 64
_B = 4096
_L = 200

_info = plsc.get_sparse_core_info()
_NC = _info.num_cores
_NS = _info.num_subcores
_NW = _NC * _NS
_CHUNK = 128
_CPW = _L
_NBUF = 4

_SCALE = math.sqrt(_EMB)

_LT = _L // 8
_ET = _EMB // 8


def _body(tok_hbm, table_hbm, out_hbm, idx_v, rows, buf, gsem, osem):
    wid = lax.axis_index("s") * _NC + lax.axis_index("c")
    pltpu.sync_copy(tok_hbm.at[:, wid], idx_v)
    iota = lax.iota(jnp.int32, 16)

    def gstart(j, b):
        pltpu.async_copy(table_hbm.at[idx_v.at[j // 8, j % 8]], rows[b], gsem[b])

    def gwait(j, b):
        pltpu.make_async_copy(
            table_hbm.at[idx_v.at[j // 8, j % 8]], rows[b], gsem[b]
        ).wait()

    def ostart(j, b):
        pltpu.async_copy(buf[b], out_hbm.at[pl.ds(j * _ET, _ET), wid], osem[b])

    def owait(j, b):
        pltpu.make_async_copy(
            buf[b], out_hbm.at[pl.ds(j * _ET, _ET), wid], osem[b]
        ).wait()

    def transpose_scale(b):
        @plsc.parallel_loop(0, _EMB, unroll=8)
        def _(e):
            col_idx = jnp.full((16,), 0, jnp.int32) + e
            for kg in range(_CHUNK // 16):
                row_idx = kg * 16 + iota
                v = plsc.load_gather(rows[b], [row_idx, col_idx])
                buf[b][e // 8, e % 8, pl.ds(kg * 16, 16)] = v * _SCALE

    gstart(0, 0)
    gstart(1, 1)

    for b in range(_NBUF):
        j = b
        if j >= 2:
            owait(j - 2, (b - 2) % _NBUF)
        gstart(j + 2, (b + 2) % _NBUF)
        gwait(j, b)
        transpose_scale(b)
        ostart(j, b)

    @pl.loop(0, (_CPW - 2 * _NBUF) // _NBUF)
    def _(r):
        j0 = _NBUF + r * _NBUF
        for b in range(_NBUF):
            j = j0 + b
            owait(j - 2, (b - 2) % _NBUF)
            gstart(j + 2, (b + 2) % _NBUF)
            gwait(j, b)
            transpose_scale(b)
            ostart(j, b)

    for b in range(_NBUF):
        j = _CPW - _NBUF + b
        owait(j - 2, (b - 2) % _NBUF)
        if j + 2 < _CPW:
            gstart(j + 2, (b + 2) % _NBUF)
        gwait(j, b)
        transpose_scale(b)
        ostart(j, b)

    owait(_CPW - 2, (_NBUF - 2) % _NBUF)
    owait(_CPW - 1, _NBUF - 1)


_mesh = plsc.VectorSubcoreMesh(core_axis_name="c", subcore_axis_name="s")

_gather = functools.partial(
    pl.kernel,
    mesh=_mesh,
    out_type=jax.ShapeDtypeStruct((_L * _ET, _NW, 8, _CHUNK), jnp.float32),
    scratch_types=[
        pltpu.VMEM((_LT, 8, _CHUNK), jnp.int32),
        [pltpu.VMEM((_CHUNK, _EMB), jnp.float32) for _ in range(_NBUF)],
        [pltpu.VMEM((_ET, 8, _CHUNK), jnp.float32) for _ in range(_NBUF)],
        [pltpu.SemaphoreType.DMA for _ in range(_NBUF)],
        [pltpu.SemaphoreType.DMA for _ in range(_NBUF)],
    ],
    compiler_params=pltpu.CompilerParams(
        use_tc_tiling_on_sc=False, needs_layout_passes=False
    ),
)(_body)


def kernel(tokens, table):
    tok = (
        tokens.astype(jnp.int32)
        .T.reshape(_LT, 8, _NW, _CHUNK)
        .transpose(0, 2, 1, 3)
    )
    out = _gather(tok, table)
    return (
        out.reshape(_L, _ET, _NW, 8, _CHUNK)
        .transpose(2, 4, 0, 1, 3)
        .reshape(_B, _L, _EMB)
    )
